# baseline (speedup 1.0000x reference)
import functools

import jax
import jax.numpy as jnp
from jax import lax
from jax.experimental import pallas as pl
from jax.experimental.pallas import tpu as pltpu
from jax.experimental.pallas import tpu_sc as plsc

VOCAB = 1000000
EMB = 32
BATCH = 16384
HIST = 50

NC, NS = 2, 16
NW = NC * NS
BT = BATCH // 128
TPW = BT // NW
L = 16

_mesh = plsc.VectorSubcoreMesh(core_axis_name="c", subcore_axis_name="s")


@functools.partial(
    pl.kernel,
    out_type=jax.ShapeDtypeStruct((HIST, EMB // 8, BT, 8, 128), jnp.float32),
    mesh=_mesh,
    scratch_types=[
        pltpu.VMEM((2, TPW, 128), jnp.int32),
        pltpu.VMEM((2, TPW, 128, EMB), jnp.float32),
        pltpu.VMEM((TPW, EMB // 8, 8, 128), jnp.float32),
        [pltpu.SemaphoreType.DMA] * 2,
        [pltpu.SemaphoreType.DMA] * TPW,
        [pltpu.SemaphoreType.DMA] * TPW,
    ],
    compiler_params=pltpu.CompilerParams(use_tc_tiling_on_sc=False,
                                         needs_layout_passes=False),
)
def _gather_kernel(xt_hbm, table_hbm, out_hbm, idxb, rowsb, tileb,
                   isem, gsem, wsem):
    wid = lax.axis_index("s") * NC + lax.axis_index("c")
    bt0 = wid * TPW
    lane = lax.iota(jnp.int32, L)
    j16s = [jb * L + lane for jb in range(8)]

    def fire_idx(h, ph):
        pltpu.async_copy(xt_hbm.at[h, pl.ds(bt0, TPW), :], idxb.at[ph],
                         isem[ph])

    def drain_idx(ph):
        pltpu.make_async_copy(xt_hbm.at[0, pl.ds(0, TPW), :], idxb.at[ph],
                              isem[ph]).wait()

    def drain_write(bt):
        for fg in range(EMB // 8):
            pltpu.make_async_copy(tileb.at[bt, fg],
                                  out_hbm.at[0, fg, 0], wsem[bt]).wait()

    def transpose(ph, bt):
        src = rowsb.at[ph, bt]
        for f in range(EMB):
            fvec = jnp.full((L,), f, dtype=jnp.int32)
            for jb in range(8):
                vals = plsc.load_gather(src, [j16s[jb], fvec])
                tileb[bt, f // 8, f % 8, pl.ds(jb * L, L)] = vals

    def half(hh, ph):
        h = hh * 2 + ph
        drain_idx(ph)
        for bt in range(TPW):
            pltpu.async_copy(table_hbm.at[idxb.at[ph, bt]],
                             rowsb.at[ph, bt], gsem[bt])
        for bt in range(TPW):
            pltpu.make_async_copy(table_hbm.at[idxb.at[ph, bt]],
                                  rowsb.at[ph, bt], gsem[bt]).wait()
            if ph == 1:
                drain_write(bt)
            else:
                @pl.when(hh > 0)
                def _():
                    drain_write(bt)
            transpose(ph, bt)
            for fg in range(EMB // 8):
                pltpu.async_copy(tileb.at[bt, fg],
                                 out_hbm.at[h, fg, bt0 + bt], wsem[bt])

        @pl.when(h + 2 < HIST)
        def _():
            fire_idx(h + 2, ph)

    def body(hh, carry):
        half(hh, 0)
        half(hh, 1)
        return 0

    fire_idx(0, 0)
    fire_idx(1, 1)
    lax.fori_loop(0, HIST // 2, body, 0)
    for bt in range(TPW):
        drain_write(bt)


def kernel(x, table):
    xt = x.T.astype(jnp.int32).reshape(HIST, BT, 128)
    out6 = _gather_kernel(xt, table)
    return out6.transpose(2, 4, 0, 1, 3).reshape(BATCH, HIST, EMB)


# speedup vs baseline: 1.3162x; 1.3162x over previous
import functools

import jax
import jax.numpy as jnp
from jax import lax
from jax.experimental import pallas as pl
from jax.experimental.pallas import tpu as pltpu
from jax.experimental.pallas import tpu_sc as plsc

VOCAB = 1000000
EMB = 32
BATCH = 16384
HIST = 50

NC, NS = 2, 16
NW = NC * NS
COLS = BATCH // NW
RH = 2
NSTEP = HIST // RH
NBUF = 2
L = 16

_mesh = plsc.VectorSubcoreMesh(core_axis_name="c", subcore_axis_name="s")


@functools.partial(
    pl.kernel,
    out_type=jax.ShapeDtypeStruct((HIST, BATCH, EMB), jnp.float32),
    mesh=_mesh,
    scratch_types=[
        pltpu.VMEM((NBUF, RH, COLS), jnp.int32),
        pltpu.VMEM((NBUF, RH, COLS, EMB), jnp.float32),
        [pltpu.SemaphoreType.DMA] * NBUF,
        [pltpu.SemaphoreType.DMA] * NBUF,
    ],
    compiler_params=pltpu.CompilerParams(use_tc_tiling_on_sc=False),
)
def _gather_kernel(xt_hbm, table_hbm, out_hbm, idx_v, rows_v,
                   gsem, wsem):
    wid = lax.axis_index("s") * NC + lax.axis_index("c")
    c0 = wid * COLS

    gathers = [None] * NBUF
    writes = [None] * NBUF

    def start_gather(i, b):
        pltpu.sync_copy(xt_hbm.at[pl.ds(i * RH, RH), pl.ds(c0, COLS)],
                        idx_v.at[b])
        gathers[b] = [
            pltpu.async_copy(table_hbm.at[idx_v.at[b, r]], rows_v.at[b, r],
                             gsem[b])
            for r in range(RH)
        ]

    for b in range(NBUF):
        start_gather(b, b)
    for i in range(NSTEP):
        b = i % NBUF
        for g in gathers[b]:
            g.wait()
        writes[b] = pltpu.async_copy(
            rows_v.at[b],
            out_hbm.at[pl.ds(i * RH, RH), pl.ds(c0, COLS), :],
            wsem[b])
        j = i + NBUF
        if j < NSTEP:
            writes[b].wait()
            start_gather(j, b)
    for b in range(NBUF):
        writes[b].wait()


def kernel(x, table):
    out = _gather_kernel(x.T.astype(jnp.int32), table)
    return out.transpose(1, 0, 2)


# speedup vs baseline: 1.3165x; 1.0002x over previous
import functools

import jax
import jax.numpy as jnp
from jax import lax
from jax.experimental import pallas as pl
from jax.experimental.pallas import tpu as pltpu
from jax.experimental.pallas import tpu_sc as plsc

VOCAB = 1000000
EMB = 32
BATCH = 16384
HIST = 50

NC, NS = 2, 16
NW = NC * NS
COLS = BATCH // NW
RH = 2
NSTEP = HIST // RH
NBUF = 2

_mesh = plsc.VectorSubcoreMesh(core_axis_name="c", subcore_axis_name="s")


@functools.partial(
    pl.kernel,
    out_type=jax.ShapeDtypeStruct((HIST, BATCH, EMB), jnp.float32),
    mesh=_mesh,
    scratch_types=[
        pltpu.VMEM((NBUF, RH, COLS), jnp.int32),
        pltpu.VMEM((NBUF, RH, COLS, EMB), jnp.float32),
        [pltpu.SemaphoreType.DMA] * NBUF,
        [pltpu.SemaphoreType.DMA] * NBUF,
    ],
    compiler_params=pltpu.CompilerParams(use_tc_tiling_on_sc=False),
)
def _gather_kernel(xt_hbm, table_hbm, out_hbm, idx_v, rows_v,
                   gsem, wsem):
    wid = lax.axis_index("s") * NC + lax.axis_index("c")
    c0 = wid * COLS

    gathers = [None] * NBUF
    writes = [None] * NBUF

    def start_gather(i, b):
        pltpu.sync_copy(xt_hbm.at[pl.ds(i * RH, RH), pl.ds(c0, COLS)],
                        idx_v.at[b])
        gathers[b] = [
            pltpu.async_copy(table_hbm.at[idx_v.at[b, r]], rows_v.at[b, r],
                             gsem[b])
            for r in range(RH)
        ]

    for b in range(NBUF):
        start_gather(b, b)
    for i in range(NSTEP):
        b = i % NBUF
        for g in gathers[b]:
            g.wait()
        writes[b] = pltpu.async_copy(
            rows_v.at[b],
            out_hbm.at[pl.ds(i * RH, RH), pl.ds(c0, COLS), :],
            wsem[b])
        j = i + NBUF
        if j < NSTEP:
            writes[b].wait()
            start_gather(j, b)
    for b in range(NBUF):
        writes[b].wait()


def kernel(x, table):
    out = _gather_kernel(x.T.astype(jnp.int32), table)
    return out.transpose(1, 0, 2)
